# Initial kernel scaffold; baseline (speedup 1.0000x reference)
import functools

import jax
import jax.numpy as jnp
from jax import lax
from jax.experimental import pallas as pl
from jax.experimental.pallas import tpu as pltpu
from jax.experimental.pallas import tpu_sc as plsc

N = 10000
D = 128
E = 320000
NPAD = 10240
NC = 2
NS = 16
NW = NC * NS
EB = 80
EPW = E // NW
BPW = EPW // EB
RPW = NPAD // NS


def _sc_agg_body(table, ei, z2d, z1d, ones_h, aggp, cntp,
                 src_buf, dst_buf, rows, ones_v, acc, cnt, sem):
    cid = lax.axis_index("c")
    sid = lax.axis_index("s")
    wid = sid * NC + cid
    r0 = sid * RPW
    pltpu.sync_copy(z2d.at[pl.ds(r0, RPW)], acc.at[pl.ds(r0, RPW)])
    pltpu.sync_copy(z1d.at[pl.ds(r0, RPW)], cnt.at[pl.ds(r0, RPW)])
    b0 = wid * BPW
    pltpu.sync_copy(ei.at[0, pl.ds(b0, BPW)], src_buf)
    pltpu.sync_copy(ei.at[1, pl.ds(b0, BPW)], dst_buf)
    pltpu.sync_copy(ones_h, ones_v)
    plsc.subcore_barrier()

    def body(j, carry):
        pltpu.async_copy(table.at[src_buf.at[j]], rows, sem).wait()
        pltpu.sync_copy(rows, acc.at[dst_buf.at[j]], add=True)
        pltpu.sync_copy(ones_v, cnt.at[dst_buf.at[j]], add=True)
        return carry

    lax.fori_loop(0, BPW, body, 0)
    plsc.subcore_barrier()
    pltpu.sync_copy(acc.at[pl.ds(r0, RPW)], aggp.at[cid, pl.ds(r0, RPW)])
    pltpu.sync_copy(cnt.at[pl.ds(r0, RPW)], cntp.at[cid, pl.ds(r0, RPW)])


_sc_aggregate = functools.partial(
    pl.kernel,
    out_type=(jax.ShapeDtypeStruct((NC, NPAD, D), jnp.float32),
              jax.ShapeDtypeStruct((NC, NPAD, 1), jnp.float32)),
    mesh=plsc.VectorSubcoreMesh(core_axis_name="c", subcore_axis_name="s"),
    scratch_types=[
        pltpu.VMEM((BPW, EB), jnp.int32),
        pltpu.VMEM((BPW, EB), jnp.int32),
        pltpu.VMEM((EB, D), jnp.float32),
        pltpu.VMEM((EB, 1), jnp.float32),
        pltpu.VMEM_SHARED((NPAD, D), jnp.float32),
        pltpu.VMEM_SHARED((NPAD, 1), jnp.float32),
        pltpu.SemaphoreType.DMA,
    ],
)(_sc_agg_body)


def _mm_t(a, w):
    return lax.dot_general(a, w, (((1,), (1,)), ((), ())),
                           precision=lax.Precision.HIGHEST)


def _tc1_body(aggp, cntp, x, wl, bl, wr, gamma, beta, h2):
    agg = aggp[0, :N, :] + aggp[1, :N, :]
    cnt = cntp[0, :N, :] + cntp[1, :N, :]
    mean = agg / jnp.maximum(cnt, 1.0)
    h = _mm_t(mean, wl[...]) + bl[...][None, :] + _mm_t(x[...], wr[...])
    mu = jnp.mean(h, axis=0, keepdims=True)
    var = jnp.mean((h - mu) ** 2, axis=0, keepdims=True)
    hn = (h - mu) / jnp.sqrt(var + 1e-5) * gamma[...][None, :] + beta[...][None, :]
    h2[...] = jnp.maximum(hn, 0.0)


def _tc2_body(aggp, cntp, hin, wl, bl, wr, out):
    agg = aggp[0, :N, :] + aggp[1, :N, :]
    cnt = cntp[0, :N, :] + cntp[1, :N, :]
    mean = agg / jnp.maximum(cnt, 1.0)
    out[...] = _mm_t(mean, wl[...]) + bl[...][None, :] + _mm_t(hin[...], wr[...])


def kernel(x, edge_index, Wl1, bl1, Wr1, gamma1, beta1, Wl2, bl2, Wr2):
    ei = edge_index.reshape(2, E // EB, EB)
    z2d = jnp.zeros((NPAD, D), jnp.float32)
    z1d = jnp.zeros((NPAD, 1), jnp.float32)
    ones_h = jnp.ones((EB, 1), jnp.float32)

    aggp1, cntp = _sc_aggregate(x, ei, z2d, z1d, ones_h)
    h2 = pl.pallas_call(
        _tc1_body,
        out_shape=jax.ShapeDtypeStruct((N, D), jnp.float32),
    )(aggp1, cntp, x, Wl1, bl1, Wr1, gamma1, beta1)
    aggp2, _ = _sc_aggregate(h2, ei, z2d, z1d, ones_h)
    out = pl.pallas_call(
        _tc2_body,
        out_shape=jax.ShapeDtypeStruct((N, D), jnp.float32),
    )(aggp2, cntp, h2, Wl2, bl2, Wr2)
    return out


# speedup vs baseline: 7.2729x; 7.2729x over previous
import functools

import jax
import jax.numpy as jnp
from jax import lax
from jax.experimental import pallas as pl
from jax.experimental.pallas import tpu as pltpu
from jax.experimental.pallas import tpu_sc as plsc

N = 10000
D = 128
E = 320000
NPAD = 10240
NC = 2
NS = 16
NW = NC * NS
EB = 80
EPW = E // NW
BPW = EPW // EB
RPW = NPAD // NS
RB = 1000
NB = N // RB


def _sc_agg_body(table, ei, z2d, z1d, ones_h, aggp, cntp,
                 src_buf, dst_buf, rows, ones_v, acc, cnt, sem):
    cid = lax.axis_index("c")
    sid = lax.axis_index("s")
    wid = sid * NC + cid
    r0 = sid * RPW
    pltpu.sync_copy(z2d.at[pl.ds(r0, RPW)], acc.at[pl.ds(r0, RPW)])
    pltpu.sync_copy(z1d.at[pl.ds(r0, RPW)], cnt.at[pl.ds(r0, RPW)])
    pltpu.sync_copy(ei.at[0, wid], src_buf)
    pltpu.sync_copy(ei.at[1, wid], dst_buf)
    pltpu.sync_copy(ones_h, ones_v)
    plsc.subcore_barrier()

    def body(j, carry):
        pltpu.async_copy(table.at[src_buf.at[j]], rows, sem).wait()
        pltpu.sync_copy(rows, acc.at[dst_buf.at[j]], add=True)
        pltpu.sync_copy(ones_v, cnt.at[dst_buf.at[j]], add=True)
        return carry

    lax.fori_loop(0, BPW, body, 0)
    plsc.subcore_barrier()
    pltpu.sync_copy(acc.at[pl.ds(r0, RPW)], aggp.at[cid, pl.ds(r0, RPW)])
    pltpu.sync_copy(cnt.at[pl.ds(r0, RPW)], cntp.at[cid, 0, pl.ds(r0, RPW)])


_sc_aggregate = functools.partial(
    pl.kernel,
    out_type=(jax.ShapeDtypeStruct((NC, NPAD, D), jnp.float32),
              jax.ShapeDtypeStruct((NC, 1, NPAD), jnp.float32)),
    mesh=plsc.VectorSubcoreMesh(core_axis_name="c", subcore_axis_name="s"),
    scratch_types=[
        pltpu.VMEM((BPW, EB), jnp.int32),
        pltpu.VMEM((BPW, EB), jnp.int32),
        pltpu.VMEM((EB, D), jnp.float32),
        pltpu.VMEM((EB,), jnp.float32),
        pltpu.VMEM_SHARED((NPAD, D), jnp.float32),
        pltpu.VMEM_SHARED((NPAD,), jnp.float32),
        pltpu.SemaphoreType.DMA,
    ],
)(_sc_agg_body)


def _mm_t(a, w):
    return lax.dot_general(a, w, (((1,), (1,)), ((), ())),
                           precision=lax.Precision.HIGHEST)


def _mean_from_parts(aggp, cntp):
    agg = aggp[0] + aggp[1]
    cnt = cntp[0] + cntp[1]
    return agg / jnp.maximum(cnt, 1.0)


def _lin_body(aggp, cntp, xin, wl, bl, wr, h_out, s1_out, s2_out):
    mean = _mean_from_parts(aggp, cntp)
    h = _mm_t(mean, wl[...]) + bl[...][None, :] + _mm_t(xin[...], wr[...])
    h_out[...] = h

    @pl.when(pl.program_id(0) == 0)
    def _init():
        s1_out[...] = jnp.zeros_like(s1_out)
        s2_out[...] = jnp.zeros_like(s2_out)

    s1_out[...] += jnp.sum(h, axis=0, keepdims=True)
    s2_out[...] += jnp.sum(h * h, axis=0, keepdims=True)


def _bn_relu_body(h, s1, s2, gamma, beta, h2):
    mu = s1[...] / N
    var = s2[...] / N - mu * mu
    inv = gamma[...][None, :] / jnp.sqrt(var + 1e-5)
    h2[...] = jnp.maximum((h[...] - mu) * inv + beta[...][None, :], 0.0)


def _lin2_body(aggp, cntp, hin, wl, bl, wr, out):
    mean = _mean_from_parts(aggp, cntp)
    out[...] = _mm_t(mean, wl[...]) + bl[...][None, :] + _mm_t(hin[...], wr[...])


_ROW = pl.BlockSpec((RB, D), lambda i: (i, 0))
_AGGP = pl.BlockSpec((NC, RB, D), lambda i: (0, i, 0))
_CNTP = pl.BlockSpec((NC, RB, 1), lambda i: (0, i, 0))
_WMAT = pl.BlockSpec((D, D), lambda i: (0, 0))
_WVEC = pl.BlockSpec((D,), lambda i: (0,))
_STAT = pl.BlockSpec((1, D), lambda i: (0, 0))


def kernel(x, edge_index, Wl1, bl1, Wr1, gamma1, beta1, Wl2, bl2, Wr2):
    ei = edge_index.reshape(2, NW, BPW, EB)
    z2d = jnp.zeros((NPAD, D), jnp.float32)
    z1d = jnp.zeros((NPAD,), jnp.float32)
    ones_h = jnp.ones((EB,), jnp.float32)

    aggp1, cntp = _sc_aggregate(x, ei, z2d, z1d, ones_h)
    cntp = cntp.reshape(NC, NPAD, 1)

    h, s1, s2 = pl.pallas_call(
        _lin_body,
        grid=(NB,),
        in_specs=[_AGGP, _CNTP, _ROW, _WMAT, _WVEC, _WMAT],
        out_specs=[_ROW, _STAT, _STAT],
        out_shape=[jax.ShapeDtypeStruct((N, D), jnp.float32),
                   jax.ShapeDtypeStruct((1, D), jnp.float32),
                   jax.ShapeDtypeStruct((1, D), jnp.float32)],
    )(aggp1, cntp, x, Wl1, bl1, Wr1)

    h2 = pl.pallas_call(
        _bn_relu_body,
        grid=(NB,),
        in_specs=[_ROW, _STAT, _STAT, _WVEC, _WVEC],
        out_specs=_ROW,
        out_shape=jax.ShapeDtypeStruct((N, D), jnp.float32),
    )(h, s1, s2, gamma1, beta1)

    aggp2, _ = _sc_aggregate(h2, ei, z2d, z1d, ones_h)

    out = pl.pallas_call(
        _lin2_body,
        grid=(NB,),
        in_specs=[_AGGP, _CNTP, _ROW, _WMAT, _WVEC, _WMAT],
        out_specs=_ROW,
        out_shape=jax.ShapeDtypeStruct((N, D), jnp.float32),
    )(aggp2, cntp, h2, Wl2, bl2, Wr2)
    return out


# speedup vs baseline: 8.5876x; 1.1808x over previous
import functools

import jax
import jax.numpy as jnp
from jax import lax
from jax.experimental import pallas as pl
from jax.experimental.pallas import tpu as pltpu
from jax.experimental.pallas import tpu_sc as plsc

N = 10000
D = 128
DH = D // 2
E = 320000
NPAD = 10240
NC = 2
NS = 16
NW = NC * NS
EB = 125
EPW = E // NW
BPW = EPW // EB
KPW = BPW // 2
RPW = NPAD // NS
RB = 1000
NB = N // RB


def _agg_phase(table, src_buf, dst_buf, rows0, rows1, ones_v, acc, cnt,
               g0, g1, with_cnt):
    pltpu.async_copy(table.at[src_buf.at[0]], rows0, g0)

    def body(k, carry):
        j0 = 2 * k
        j1 = j0 + 1
        pltpu.async_copy(table.at[src_buf.at[j1]], rows1, g1)
        pltpu.make_async_copy(table.at[src_buf.at[j0]], rows0, g0).wait()
        pltpu.sync_copy(rows0, acc.at[dst_buf.at[j0]], add=True)
        if with_cnt:
            pltpu.sync_copy(ones_v, cnt.at[dst_buf.at[j0]], add=True)
        jn = jnp.minimum(j0 + 2, BPW - 2)
        pltpu.async_copy(table.at[src_buf.at[jn]], rows0, g0)
        pltpu.make_async_copy(table.at[src_buf.at[j1]], rows1, g1).wait()
        pltpu.sync_copy(rows1, acc.at[dst_buf.at[j1]], add=True)
        if with_cnt:
            pltpu.sync_copy(ones_v, cnt.at[dst_buf.at[j1]], add=True)
        return carry

    lax.fori_loop(0, KPW, body, 0)
    pltpu.make_async_copy(table.at[src_buf.at[0]], rows0, g0).wait()


def _sc_agg_body(tabA, tabB, ei, z2d, z1d, ones_h, aggpA, aggpB, cntp,
                 src_buf, dst_buf, rows0, rows1, ones_v, acc, cnt, g0, g1):
    cid = lax.axis_index("c")
    sid = lax.axis_index("s")
    wid = sid * NC + cid
    r0 = sid * RPW
    pltpu.sync_copy(z2d.at[pl.ds(r0, RPW)], acc.at[pl.ds(r0, RPW)])
    pltpu.sync_copy(z1d.at[pl.ds(r0, RPW)], cnt.at[pl.ds(r0, RPW)])
    pltpu.sync_copy(ei.at[0, wid], src_buf)
    pltpu.sync_copy(ei.at[1, wid], dst_buf)
    pltpu.sync_copy(ones_h, ones_v)
    plsc.subcore_barrier()

    _agg_phase(tabA, src_buf, dst_buf, rows0, rows1, ones_v, acc, cnt,
               g0, g1, with_cnt=True)
    plsc.subcore_barrier()
    pltpu.sync_copy(acc.at[pl.ds(r0, RPW)], aggpA.at[cid, pl.ds(r0, RPW)])
    pltpu.sync_copy(cnt.at[pl.ds(r0, RPW)], cntp.at[cid, 0, pl.ds(r0, RPW)])
    pltpu.sync_copy(z2d.at[pl.ds(r0, RPW)], acc.at[pl.ds(r0, RPW)])
    plsc.subcore_barrier()

    _agg_phase(tabB, src_buf, dst_buf, rows0, rows1, ones_v, acc, cnt,
               g0, g1, with_cnt=False)
    plsc.subcore_barrier()
    pltpu.sync_copy(acc.at[pl.ds(r0, RPW)], aggpB.at[cid, pl.ds(r0, RPW)])


_sc_aggregate = functools.partial(
    pl.kernel,
    out_type=(jax.ShapeDtypeStruct((NC, NPAD, DH), jnp.float32),
              jax.ShapeDtypeStruct((NC, NPAD, DH), jnp.float32),
              jax.ShapeDtypeStruct((NC, 1, NPAD), jnp.float32)),
    mesh=plsc.VectorSubcoreMesh(core_axis_name="c", subcore_axis_name="s"),
    compiler_params=pltpu.CompilerParams(use_tc_tiling_on_sc=False),
    scratch_types=[
        pltpu.VMEM((BPW, EB), jnp.int32),
        pltpu.VMEM((BPW, EB), jnp.int32),
        pltpu.VMEM((EB, DH), jnp.float32),
        pltpu.VMEM((EB, DH), jnp.float32),
        pltpu.VMEM((EB,), jnp.float32),
        pltpu.VMEM_SHARED((NPAD, DH), jnp.float32),
        pltpu.VMEM_SHARED((NPAD,), jnp.float32),
        pltpu.SemaphoreType.DMA,
        pltpu.SemaphoreType.DMA,
    ],
)(_sc_agg_body)


def _mm_t(a, w):
    return lax.dot_general(a, w, (((1,), (1,)), ((), ())),
                           precision=lax.Precision.HIGHEST)


def _sage_lin(aggpA, aggpB, cntp, wl, bl, hA, hB, wr):
    cnt = jnp.maximum(cntp[0] + cntp[1], 1.0)
    meanA = (aggpA[0] + aggpA[1]) / cnt
    meanB = (aggpB[0] + aggpB[1]) / cnt
    return (_mm_t(meanA, wl[:, :DH]) + _mm_t(meanB, wl[:, DH:])
            + bl[...][None, :]
            + _mm_t(hA[...], wr[:, :DH]) + _mm_t(hB[...], wr[:, DH:]))


def _lin_body(aggpA, aggpB, cntp, xA, xB, wl, bl, wr, h_out, s1_out, s2_out):
    h = _sage_lin(aggpA, aggpB, cntp, wl, bl, xA, xB, wr)
    h_out[...] = h

    @pl.when(pl.program_id(0) == 0)
    def _init():
        s1_out[...] = jnp.zeros_like(s1_out)
        s2_out[...] = jnp.zeros_like(s2_out)

    s1_out[...] += jnp.sum(h, axis=0, keepdims=True)
    s2_out[...] += jnp.sum(h * h, axis=0, keepdims=True)


def _bn_relu_body(h, s1, s2, gamma, beta, h2A, h2B):
    mu = s1[...] / N
    var = s2[...] / N - mu * mu
    inv = gamma[...][None, :] / jnp.sqrt(var + 1e-5)
    h2 = jnp.maximum((h[...] - mu) * inv + beta[...][None, :], 0.0)
    h2A[...] = h2[:, :DH]
    h2B[...] = h2[:, DH:]


def _lin2_body(aggpA, aggpB, cntp, hA, hB, wl, bl, wr, out):
    out[...] = _sage_lin(aggpA, aggpB, cntp, wl, bl, hA, hB, wr)


_ROW = pl.BlockSpec((RB, D), lambda i: (i, 0))
_ROWH = pl.BlockSpec((RB, DH), lambda i: (i, 0))
_AGGP = pl.BlockSpec((NC, RB, DH), lambda i: (0, i, 0))
_CNTP = pl.BlockSpec((NC, RB, 1), lambda i: (0, i, 0))
_WMAT = pl.BlockSpec((D, D), lambda i: (0, 0))
_WVEC = pl.BlockSpec((D,), lambda i: (0,))
_STAT = pl.BlockSpec((1, D), lambda i: (0, 0))


def kernel(x, edge_index, Wl1, bl1, Wr1, gamma1, beta1, Wl2, bl2, Wr2):
    ei = edge_index.reshape(2, NW, BPW, EB)
    z2d = jnp.zeros((NPAD, DH), jnp.float32)
    z1d = jnp.zeros((NPAD,), jnp.float32)
    ones_h = jnp.ones((EB,), jnp.float32)
    xA = x[:, :DH]
    xB = x[:, DH:]

    aggpA1, aggpB1, cntp = _sc_aggregate(xA, xB, ei, z2d, z1d, ones_h)
    cntp = cntp.reshape(NC, NPAD, 1)

    h, s1, s2 = pl.pallas_call(
        _lin_body,
        grid=(NB,),
        in_specs=[_AGGP, _AGGP, _CNTP, _ROWH, _ROWH, _WMAT, _WVEC, _WMAT],
        out_specs=[_ROW, _STAT, _STAT],
        out_shape=[jax.ShapeDtypeStruct((N, D), jnp.float32),
                   jax.ShapeDtypeStruct((1, D), jnp.float32),
                   jax.ShapeDtypeStruct((1, D), jnp.float32)],
    )(aggpA1, aggpB1, cntp, xA, xB, Wl1, bl1, Wr1)

    h2A, h2B = pl.pallas_call(
        _bn_relu_body,
        grid=(NB,),
        in_specs=[_ROW, _STAT, _STAT, _WVEC, _WVEC],
        out_specs=[_ROWH, _ROWH],
        out_shape=[jax.ShapeDtypeStruct((N, DH), jnp.float32),
                   jax.ShapeDtypeStruct((N, DH), jnp.float32)],
    )(h, s1, s2, gamma1, beta1)

    aggpA2, aggpB2, _ = _sc_aggregate(h2A, h2B, ei, z2d, z1d, ones_h)

    out = pl.pallas_call(
        _lin2_body,
        grid=(NB,),
        in_specs=[_AGGP, _AGGP, _CNTP, _ROWH, _ROWH, _WMAT, _WVEC, _WMAT],
        out_specs=_ROW,
        out_shape=jax.ShapeDtypeStruct((N, D), jnp.float32),
    )(aggpA2, aggpB2, cntp, h2A, h2B, Wl2, bl2, Wr2)
    return out


# speedup vs baseline: 9.8271x; 1.1443x over previous
import functools

import jax
import jax.numpy as jnp
from jax import lax
from jax.experimental import pallas as pl
from jax.experimental.pallas import tpu as pltpu
from jax.experimental.pallas import tpu_sc as plsc

N = 10000
D = 128
DH = D // 2
E = 320000
NPAD = 10240
NC = 2
NS = 16
NW = NC * NS
EB = 125
EPW = E // NW
BPW = EPW // EB
KPW = BPW // 2
RPW = NPAD // NS
ZR = 128
RB = 2000
NB = N // RB


def _agg_phase(table, src_buf, dst_buf, rows0, rows1, ones_v, acc, cnt,
               g0, g1, with_cnt):
    pltpu.async_copy(table.at[src_buf.at[0]], rows0, g0)

    def body(k, carry):
        j0 = 2 * k
        j1 = j0 + 1
        pltpu.async_copy(table.at[src_buf.at[j1]], rows1, g1)
        pltpu.make_async_copy(table.at[src_buf.at[j0]], rows0, g0).wait()
        pltpu.sync_copy(rows0, acc.at[dst_buf.at[j0]], add=True)
        if with_cnt:
            pltpu.sync_copy(ones_v, cnt.at[dst_buf.at[j0]], add=True)
        jn = jnp.minimum(j0 + 2, BPW - 2)
        pltpu.async_copy(table.at[src_buf.at[jn]], rows0, g0)
        pltpu.make_async_copy(table.at[src_buf.at[j1]], rows1, g1).wait()
        pltpu.sync_copy(rows1, acc.at[dst_buf.at[j1]], add=True)
        if with_cnt:
            pltpu.sync_copy(ones_v, cnt.at[dst_buf.at[j1]], add=True)
        return carry

    lax.fori_loop(0, KPW, body, 0)
    pltpu.make_async_copy(table.at[src_buf.at[0]], rows0, g0).wait()


def _sc_agg_body(tabA, tabB, ei, aggpA, aggpB, cntp,
                 src_buf, dst_buf, rows0, rows1, zbuf, zcnt, ones_v,
                 acc, cnt, g0, g1):
    cid = lax.axis_index("c")
    sid = lax.axis_index("s")
    wid = sid * NC + cid
    r0 = sid * RPW
    z16 = jnp.zeros((16,), jnp.float32)
    o16 = jnp.ones((16,), jnp.float32)

    def _zfill(i, c):
        for k in range(DH // 16):
            zbuf[i, pl.ds(16 * k, 16)] = z16
        return c

    lax.fori_loop(0, ZR, _zfill, 0)

    def _zcfill(i, c):
        zcnt[pl.ds(16 * i, 16)] = z16
        return c

    lax.fori_loop(0, RPW // 16, _zcfill, 0)

    def _ofill(i, c):
        ones_v[pl.ds(16 * i, 16)] = o16
        return c

    lax.fori_loop(0, 8, _ofill, 0)

    def _zero_acc():
        for k in range(RPW // ZR):
            pltpu.sync_copy(zbuf, acc.at[pl.ds(r0 + ZR * k, ZR)])

    _zero_acc()
    pltpu.sync_copy(zcnt, cnt.at[pl.ds(r0, RPW)])
    pltpu.sync_copy(ei.at[0, wid], src_buf)
    pltpu.sync_copy(ei.at[1, wid], dst_buf)
    plsc.subcore_barrier()

    _agg_phase(tabA, src_buf, dst_buf, rows0, rows1,
               ones_v.at[pl.ds(0, EB)], acc, cnt, g0, g1, with_cnt=True)
    plsc.subcore_barrier()
    pltpu.sync_copy(acc.at[pl.ds(r0, RPW)], aggpA.at[cid, pl.ds(r0, RPW)])
    pltpu.sync_copy(cnt.at[pl.ds(r0, RPW)], cntp.at[cid, 0, pl.ds(r0, RPW)])
    _zero_acc()
    plsc.subcore_barrier()

    _agg_phase(tabB, src_buf, dst_buf, rows0, rows1,
               ones_v.at[pl.ds(0, EB)], acc, cnt, g0, g1, with_cnt=False)
    plsc.subcore_barrier()
    pltpu.sync_copy(acc.at[pl.ds(r0, RPW)], aggpB.at[cid, pl.ds(r0, RPW)])


_sc_aggregate = functools.partial(
    pl.kernel,
    out_type=(jax.ShapeDtypeStruct((NC, NPAD, DH), jnp.float32),
              jax.ShapeDtypeStruct((NC, NPAD, DH), jnp.float32),
              jax.ShapeDtypeStruct((NC, 1, NPAD), jnp.float32)),
    mesh=plsc.VectorSubcoreMesh(core_axis_name="c", subcore_axis_name="s"),
    compiler_params=pltpu.CompilerParams(use_tc_tiling_on_sc=False),
    scratch_types=[
        pltpu.VMEM((BPW, EB), jnp.int32),
        pltpu.VMEM((BPW, EB), jnp.int32),
        pltpu.VMEM((EB, DH), jnp.float32),
        pltpu.VMEM((EB, DH), jnp.float32),
        pltpu.VMEM((ZR, DH), jnp.float32),
        pltpu.VMEM((RPW,), jnp.float32),
        pltpu.VMEM((128,), jnp.float32),
        pltpu.VMEM_SHARED((NPAD, DH), jnp.float32),
        pltpu.VMEM_SHARED((NPAD,), jnp.float32),
        pltpu.SemaphoreType.DMA,
        pltpu.SemaphoreType.DMA,
    ],
)(_sc_agg_body)


def _mm_t(a, w):
    return lax.dot_general(a, w, (((1,), (1,)), ((), ())))


def _sage_lin(aggpA, aggpB, cntp, wl, bl, hA, hB, wr):
    cnt = jnp.maximum(cntp[0] + cntp[1], 1.0)
    meanA = (aggpA[0] + aggpA[1]) / cnt
    meanB = (aggpB[0] + aggpB[1]) / cnt
    return (_mm_t(meanA, wl[:, :DH]) + _mm_t(meanB, wl[:, DH:])
            + bl[...][None, :]
            + _mm_t(hA[...], wr[:, :DH]) + _mm_t(hB[...], wr[:, DH:]))


def _lin_body(aggpA, aggpB, cntp, xA, xB, wl, bl, wr, h_out, s1_out, s2_out):
    h = _sage_lin(aggpA, aggpB, cntp, wl, bl, xA, xB, wr)
    h_out[...] = h

    @pl.when(pl.program_id(0) == 0)
    def _init():
        s1_out[...] = jnp.zeros_like(s1_out)
        s2_out[...] = jnp.zeros_like(s2_out)

    s1_out[...] += jnp.sum(h, axis=0, keepdims=True)
    s2_out[...] += jnp.sum(h * h, axis=0, keepdims=True)


def _bn_relu_body(h, s1, s2, gamma, beta, h2A, h2B):
    mu = s1[...] / N
    var = s2[...] / N - mu * mu
    inv = gamma[...][None, :] / jnp.sqrt(var + 1e-5)
    h2 = jnp.maximum((h[...] - mu) * inv + beta[...][None, :], 0.0)
    h2A[...] = h2[:, :DH]
    h2B[...] = h2[:, DH:]


def _lin2_body(aggpA, aggpB, cntp, hA, hB, wl, bl, wr, out):
    out[...] = _sage_lin(aggpA, aggpB, cntp, wl, bl, hA, hB, wr)


_ROW = pl.BlockSpec((RB, D), lambda i: (i, 0))
_ROWH = pl.BlockSpec((RB, DH), lambda i: (i, 0))
_AGGP = pl.BlockSpec((NC, RB, DH), lambda i: (0, i, 0))
_CNTP = pl.BlockSpec((NC, RB, 1), lambda i: (0, i, 0))
_WMAT = pl.BlockSpec((D, D), lambda i: (0, 0))
_WVEC = pl.BlockSpec((D,), lambda i: (0,))
_STAT = pl.BlockSpec((1, D), lambda i: (0, 0))


def kernel(x, edge_index, Wl1, bl1, Wr1, gamma1, beta1, Wl2, bl2, Wr2):
    ei = edge_index.reshape(2, NW, BPW, EB)
    xA = x[:, :DH]
    xB = x[:, DH:]

    aggpA1, aggpB1, cntp = _sc_aggregate(xA, xB, ei)
    cntp = cntp.reshape(NC, NPAD, 1)

    h, s1, s2 = pl.pallas_call(
        _lin_body,
        grid=(NB,),
        in_specs=[_AGGP, _AGGP, _CNTP, _ROWH, _ROWH, _WMAT, _WVEC, _WMAT],
        out_specs=[_ROW, _STAT, _STAT],
        out_shape=[jax.ShapeDtypeStruct((N, D), jnp.float32),
                   jax.ShapeDtypeStruct((1, D), jnp.float32),
                   jax.ShapeDtypeStruct((1, D), jnp.float32)],
    )(aggpA1, aggpB1, cntp, xA, xB, Wl1, bl1, Wr1)

    h2A, h2B = pl.pallas_call(
        _bn_relu_body,
        grid=(NB,),
        in_specs=[_ROW, _STAT, _STAT, _WVEC, _WVEC],
        out_specs=[_ROWH, _ROWH],
        out_shape=[jax.ShapeDtypeStruct((N, DH), jnp.float32),
                   jax.ShapeDtypeStruct((N, DH), jnp.float32)],
    )(h, s1, s2, gamma1, beta1)

    aggpA2, aggpB2, _ = _sc_aggregate(h2A, h2B, ei)

    out = pl.pallas_call(
        _lin2_body,
        grid=(NB,),
        in_specs=[_AGGP, _AGGP, _CNTP, _ROWH, _ROWH, _WMAT, _WVEC, _WMAT],
        out_specs=_ROW,
        out_shape=jax.ShapeDtypeStruct((N, D), jnp.float32),
    )(aggpA2, aggpB2, cntp, h2A, h2B, Wl2, bl2, Wr2)
    return out
